# baseline (speedup 1.0000x reference)
import functools

import jax
import jax.numpy as jnp
from jax import lax
from jax.experimental import pallas as pl
from jax.experimental.pallas import tpu as pltpu
from jax.experimental.pallas import tpu_sc as plsc

N = 10000
E = 320000
H = 128

_NC = 2
_NS = 16
_NW = _NC * _NS
_EPT = E // _NW


_NR = 80
_NL = 128
_NP = _NR * _NL
_CHUNK = 16


_WIN = 10112


@functools.lru_cache(maxsize=1)
def _make_sc_segment():
    @functools.partial(
        pl.kernel,
        out_type=[
            jax.ShapeDtypeStruct((_NC, _NR, _NL), jnp.float32),
            jax.ShapeDtypeStruct((_NC, _NR, _NL), jnp.float32),
        ],
        mesh=plsc.VectorSubcoreMesh(core_axis_name="c", subcore_axis_name="s"),
        compiler_params=pltpu.CompilerParams(
            needs_layout_passes=False, use_tc_tiling_on_sc=False),
        scratch_types=[
            pltpu.VMEM((N,), jnp.float32),
            pltpu.VMEM((2, _WIN), jnp.int32),
            pltpu.VMEM((_NR, _NL), jnp.float32),
            pltpu.VMEM((_NR, _NL), jnp.float32),
            pltpu.VMEM((_NR,), jnp.int32),
            pltpu.VMEM_SHARED((_NR, _NL), jnp.float32),
            pltpu.VMEM_SHARED((_NR, _NL), jnp.float32),
        ],
    )
    def _sc_segment(x_hbm, ei_hbm, zeros_hbm, iota_hbm, s_out, d_out,
                    x_v, ei_v, acc_s, acc_d, rows_v, sh_s, sh_d):
        c = lax.axis_index("c")
        s = lax.axis_index("s")
        base = (c * _NS + s) * _EPT
        off0 = lax.rem(base, _NL)
        awin0 = base - off0

        pltpu.sync_copy(x_hbm, x_v)
        pltpu.sync_copy(ei_hbm.at[:, pl.ds(awin0, _WIN)], ei_v)
        pltpu.sync_copy(zeros_hbm, acc_s)
        pltpu.sync_copy(zeros_hbm, acc_d)
        pltpu.sync_copy(iota_hbm, rows_v)

        @pl.when(s == 0)
        def _():
            pltpu.sync_copy(zeros_hbm, sh_s)
            pltpu.sync_copy(zeros_hbm, sh_d)

        ones16 = jnp.ones((_CHUNK,), jnp.float32)
        zeros16i = jnp.zeros((_CHUNK,), jnp.int32)
        ones16i = jnp.ones((_CHUNK,), jnp.int32)
        iota16 = lax.iota(jnp.int32, _CHUNK)

        _UB = 5

        @plsc.parallel_loop(0, _EPT // _CHUNK, _UB, unroll=4)
        def body(j):
            staged = []
            for k in range(_UB):
                lane = off0 + (j + k) * _CHUNK + iota16
                sidx = plsc.load_gather(ei_v, [zeros16i, lane])
                didx = plsc.load_gather(ei_v, [ones16i, lane])
                vals = plsc.load_gather(x_v, [sidx])
                staged.append((lax.shift_right_logical(didx, 7),
                               lax.bitwise_and(didx, 127), vals))
            for row, col, vals in staged:
                plsc.addupdate_scatter(acc_s, [row, col], vals)
            for row, col, _ in staged:
                plsc.addupdate_scatter(acc_d, [row, col], ones16)

        plsc.subcore_barrier()
        pltpu.sync_copy(acc_s, sh_s.at[rows_v], add=True)
        pltpu.sync_copy(acc_d, sh_d.at[rows_v], add=True)
        plsc.subcore_barrier()

        @pl.when(s == 0)
        def _():
            pltpu.sync_copy(sh_s, s_out.at[c])
            pltpu.sync_copy(sh_d, d_out.at[c])

    return _sc_segment


_TROWS = 16
_TBLK = _TROWS * _NL


def _tc_body(x_ref, s_ref, d_ref, wenc_ref, benc_ref, wself_ref, wneigh_ref,
             bproc_ref, wdec_ref, bdec_ref, wterm_ref, bterm_ref,
             y_ref, h_ref, t_ref):
    w0 = wenc_ref[0:1, :]
    benc = benc_ref[...]
    u = jnp.dot(w0, wself_ref[...], preferred_element_type=jnp.float32)
    v = jnp.dot(w0, wneigh_ref[...], preferred_element_type=jnp.float32)
    cbias = (jnp.dot(benc, wself_ref[...], preferred_element_type=jnp.float32)
             + bproc_ref[...])
    cflag = jnp.dot(benc, wneigh_ref[...], preferred_element_type=jnp.float32)
    wd = wdec_ref[...]
    wd_h = wd[:H, :]
    alpha = jnp.dot(w0, wd[H:, :], preferred_element_type=jnp.float32)[0, 0]
    gamma = (jnp.dot(benc, wd[H:, :], preferred_element_type=jnp.float32)[0, 0]
             + bdec_ref[0, 0])

    ssum = s_ref[0] + s_ref[1]
    dsum = d_ref[0] + d_ref[1]
    m_rows = ssum / jnp.maximum(dsum, 1.0)
    f_rows = (dsum > 0.0).astype(jnp.float32)
    mT = jnp.transpose(m_rows)
    fT = jnp.transpose(f_rows)
    xT = jnp.transpose(x_ref[...])

    sub_iota = lax.broadcasted_iota(jnp.int32, (_NL, 1), 0)

    bmax = None
    bsum = None
    _FULL = N // _NL
    for r in range(_FULL + 1):
        nrow = min(_NL, N - r * _NL)
        m_col = mT[:, r:r + 1]
        f_col = fT[:, r:r + 1]
        x_col = xT[:, r:r + 1]
        tile = jnp.maximum(
            x_col * u + m_col * v + f_col * cflag + cbias, 0.0)
        h_ref[r * _NL:r * _NL + nrow, :] = tile[:nrow, :]
        yl = (jnp.dot(tile, wd_h, preferred_element_type=jnp.float32)
              + x_col * alpha + gamma)
        y_ref[r * _NL:r * _NL + nrow, :] = jax.nn.sigmoid(yl)[:nrow, :]
        if nrow == _NL:
            tile0 = tile
        else:
            tile0 = jnp.where(sub_iota < nrow, tile, 0.0)
        tmax = jnp.max(tile0, axis=0, keepdims=True)
        tsum = jnp.sum(tile0, axis=0, keepdims=True)
        bmax = tmax if bmax is None else jnp.maximum(bmax, tmax)
        bsum = tsum if bsum is None else bsum + tsum

    wt = wterm_ref[...]
    tv = (jnp.dot(bmax, wt[:H, :], preferred_element_type=jnp.float32)
          + jnp.dot(bsum / N, wt[H:, :], preferred_element_type=jnp.float32))
    t_ref[...] = jax.nn.sigmoid(tv + bterm_ref[...])


def _tc_call(x, s2, d2, W_enc, b_enc, W_self, W_neigh, b_proc,
             W_dec, b_dec, W_term, b_term):
    return pl.pallas_call(
        _tc_body,
        out_shape=[
            jax.ShapeDtypeStruct((N, 1), jnp.float32),
            jax.ShapeDtypeStruct((N, H), jnp.float32),
            jax.ShapeDtypeStruct((1, 1), jnp.float32),
        ],
    )(x, s2, d2, W_enc, b_enc, W_self, W_neigh, b_proc,
      W_dec, b_dec, W_term, b_term)


def kernel(x, edge_index, W_enc, b_enc, W_self, W_neigh, b_proc,
           W_dec, b_dec, W_term, b_term):
    zeros_np = jnp.zeros((_NR, _NL), jnp.float32)
    iota_nr = jnp.arange(_NR, dtype=jnp.int32)
    x_flat = x.reshape(N)
    s2, d2 = _make_sc_segment()(x_flat, edge_index, zeros_np, iota_nr)
    x_grid = jnp.pad(x_flat, (0, _NP - N)).reshape(_NR, _NL)
    y, h, t = _tc_call(
        x_grid, s2, d2, W_enc,
        b_enc.reshape(1, H), W_self, W_neigh, b_proc.reshape(1, H),
        W_dec, b_dec.reshape(1, 1), W_term, b_term.reshape(1, 1))
    return y, h, t.reshape(1)
